# baseline (speedup 1.0000x reference)
import functools

import jax
import jax.numpy as jnp
from jax import lax
from jax.experimental import pallas as pl
from jax.experimental.pallas import tpu as pltpu
from jax.experimental.pallas import tpu_sc as plsc

BATCH = 4
SEQ = 2048
D_MODEL = 1024
SCALE = 32.0

NUM_CORES = 2
NUM_SUBCORES = 16
NW = NUM_CORES * NUM_SUBCORES
POS_PER_W = SEQ // NW
CHUNK = 8
NG = POS_PER_W // CHUNK
NB = 3
LANES = 16


_mesh = plsc.VectorSubcoreMesh(core_axis_name="c", subcore_axis_name="s")


@functools.partial(
    pl.kernel,
    mesh=_mesh,
    out_type=jax.ShapeDtypeStruct((BATCH, SEQ, D_MODEL), jnp.float32),
    scratch_types=[
        pltpu.VMEM((BATCH, POS_PER_W), jnp.int32),
        pltpu.VMEM((NB, BATCH, CHUNK, D_MODEL), jnp.float32),
        pltpu.VMEM((NB, CHUNK, D_MODEL), jnp.float32),
        pltpu.SemaphoreType.DMA,
        pltpu.SemaphoreType.DMA,
        pltpu.SemaphoreType.DMA,
    ],
)
def _emb_kernel(x_hbm, table_hbm, pe_hbm, out_hbm, idx_v, rows_v, pe_v,
                gsem, psem, ssem):
    wid = lax.axis_index("s") * NUM_CORES + lax.axis_index("c")
    p0 = wid * POS_PER_W

    def fire_gathers(g):
        slot = g % NB
        cps = []
        for b in range(BATCH):
            src = table_hbm.at[idx_v.at[b, pl.ds(g * CHUNK, CHUNK)]]
            cps.append(pltpu.async_copy(src, rows_v.at[slot, b], gsem))
        return cps

    def fire_pe(g):
        src = pe_hbm.at[pl.ds(p0 + g * CHUNK, CHUNK)]
        return pltpu.async_copy(src, pe_v.at[g % NB], psem)

    pe_cp = [fire_pe(0), fire_pe(1)]
    x_cp = [pltpu.async_copy(x_hbm.at[b, pl.ds(p0, POS_PER_W)], idx_v.at[b],
                             gsem) for b in range(BATCH)]
    for cp in x_cp:
        cp.wait()
    g_cp = [fire_gathers(0), fire_gathers(1)]
    s_cp = []

    for g in range(NG):
        pe_cp[g].wait()
        if g + 2 < NG:
            pe_cp.append(fire_pe(g + 2))
        for cp in g_cp[g]:
            cp.wait()
        if g >= 1:
            for cp in s_cp[g - 1]:
                cp.wait()
        if g + 2 < NG:
            g_cp.append(fire_gathers(g + 2))

        slot = g % NB
        rb = rows_v.at[slot]
        pb = pe_v.at[slot]

        def body(j, carry):
            for r in range(CHUNK):
                pv = pb[r, pl.ds(j * LANES, LANES)]
                for b in range(BATCH):
                    sl = (b, r, pl.ds(j * LANES, LANES))
                    rb[sl] = rb[sl] * SCALE + pv
            return carry

        lax.fori_loop(0, D_MODEL // LANES, body, 0)

        cps = []
        for b in range(BATCH):
            dst = out_hbm.at[b, pl.ds(p0 + g * CHUNK, CHUNK)]
            cps.append(pltpu.async_copy(rows_v.at[slot, b], dst, ssem))
        s_cp.append(cps)

    for cp in s_cp[NG - 1]:
        cp.wait()


def kernel(x, table, pos_enc):
    return _emb_kernel(x.astype(jnp.int32), table, pos_enc)


# speedup vs baseline: 1.0452x; 1.0452x over previous
import functools

import jax
import jax.numpy as jnp
from jax import lax
from jax.experimental import pallas as pl
from jax.experimental.pallas import tpu as pltpu
from jax.experimental.pallas import tpu_sc as plsc

BATCH = 4
SEQ = 2048
D_MODEL = 1024
SCALE = 32.0

NUM_CORES = 2
NUM_SUBCORES = 16
NW = NUM_CORES * NUM_SUBCORES
POS_PER_W = SEQ // NW
CHUNK = 8
NG = POS_PER_W // CHUNK
GROUP_ROWS = BATCH * CHUNK
NB = 3
LANES = 16


_mesh = plsc.VectorSubcoreMesh(core_axis_name="c", subcore_axis_name="s")


@functools.partial(
    pl.kernel,
    mesh=_mesh,
    out_type=jax.ShapeDtypeStruct((BATCH, SEQ, D_MODEL), jnp.float32),
    scratch_types=[
        pltpu.VMEM((NG, GROUP_ROWS), jnp.int32),
        pltpu.VMEM((NB, GROUP_ROWS, D_MODEL), jnp.float32),
        pltpu.VMEM((NB, CHUNK, D_MODEL), jnp.float32),
        pltpu.SemaphoreType.DMA,
        pltpu.SemaphoreType.DMA,
        pltpu.SemaphoreType.DMA,
    ],
)
def _emb_kernel(x_hbm, table_hbm, pe_hbm, out_hbm, idx_v, rows_v, pe_v,
                gsem, psem, ssem):
    wid = lax.axis_index("s") * NUM_CORES + lax.axis_index("c")
    p0 = wid * POS_PER_W

    def fire_pe(g):
        src = pe_hbm.at[pl.ds(p0 + g * CHUNK, CHUNK)]
        return pltpu.async_copy(src, pe_v.at[g % NB], psem)

    pe_cp = [fire_pe(0), fire_pe(1)]

    x_cp = []
    for g in range(NG):
        for b in range(BATCH):
            src = x_hbm.at[b, pl.ds(p0 + g * CHUNK, CHUNK)]
            dst = idx_v.at[g, pl.ds(b * CHUNK, CHUNK)]
            x_cp.append(pltpu.async_copy(src, dst, gsem))
    for cp in x_cp:
        cp.wait()

    def fire_gather(g):
        src = table_hbm.at[idx_v.at[g]]
        return pltpu.async_copy(src, rows_v.at[g % NB], gsem)

    g_cp = [fire_gather(0), fire_gather(1)]
    s_cp = []

    for g in range(NG):
        pe_cp[g].wait()
        if g + 2 < NG:
            pe_cp.append(fire_pe(g + 2))
        g_cp[g].wait()
        if g + 2 < NG:
            if g >= 1:
                for cp in s_cp[g - 1]:
                    cp.wait()
            g_cp.append(fire_gather(g + 2))

        slot = g % NB
        rb = rows_v.at[slot]
        pb = pe_v.at[slot]

        def body(j, carry):
            for r in range(CHUNK):
                pv = pb[r, pl.ds(j * LANES, LANES)]
                for b in range(BATCH):
                    sl = (b * CHUNK + r, pl.ds(j * LANES, LANES))
                    rb[sl] = rb[sl] * SCALE + pv
            return carry

        lax.fori_loop(0, D_MODEL // LANES, body, 0)

        cps = []
        for b in range(BATCH):
            src = rows_v.at[slot, pl.ds(b * CHUNK, CHUNK)]
            dst = out_hbm.at[b, pl.ds(p0 + g * CHUNK, CHUNK)]
            cps.append(pltpu.async_copy(src, dst, ssem))
        s_cp.append(cps)

    for g in range(NG - 3, NG):
        for cp in s_cp[g]:
            cp.wait()


def kernel(x, table, pos_enc):
    return _emb_kernel(x.astype(jnp.int32), table, pos_enc)


# speedup vs baseline: 1.0460x; 1.0007x over previous
import functools

import jax
import jax.numpy as jnp
from jax import lax
from jax.experimental import pallas as pl
from jax.experimental.pallas import tpu as pltpu
from jax.experimental.pallas import tpu_sc as plsc

BATCH = 4
SEQ = 2048
D_MODEL = 1024
SCALE = 32.0

NUM_CORES = 2
NUM_SUBCORES = 16
NW = NUM_CORES * NUM_SUBCORES
POS_PER_W = SEQ // NW
CHUNK = 8
NG = POS_PER_W // CHUNK
GROUP_ROWS = BATCH * CHUNK
NB = 3
LANES = 16


_mesh = plsc.VectorSubcoreMesh(core_axis_name="c", subcore_axis_name="s")


@functools.partial(
    pl.kernel,
    mesh=_mesh,
    out_type=jax.ShapeDtypeStruct((BATCH, SEQ, D_MODEL), jnp.float32),
    scratch_types=[
        pltpu.VMEM((NG, GROUP_ROWS), jnp.int32),
        pltpu.VMEM((NB, GROUP_ROWS, D_MODEL), jnp.float32),
        pltpu.VMEM((NB, CHUNK, D_MODEL), jnp.float32),
        pltpu.SemaphoreType.DMA,
        pltpu.SemaphoreType.DMA,
        pltpu.SemaphoreType.DMA,
    ],
)
def _emb_kernel(x_hbm, table_hbm, pe_hbm, out_hbm, idx_v, rows_v, pe_v,
                gsem, psem, ssem):
    wid = lax.axis_index("s") * NUM_CORES + lax.axis_index("c")
    p0 = wid * POS_PER_W

    def fire_pe(g):
        src = pe_hbm.at[pl.ds(p0 + g * CHUNK, CHUNK)]
        return pltpu.async_copy(src, pe_v.at[g % NB], psem)

    pe_cp = [fire_pe(0), fire_pe(1)]

    x_cp = []
    for g in range(NG):
        for b in range(BATCH):
            src = x_hbm.at[b, pl.ds(p0 + g * CHUNK, CHUNK)]
            dst = idx_v.at[g, pl.ds(b * CHUNK, CHUNK)]
            x_cp.append(pltpu.async_copy(src, dst, gsem))

    def fire_gather(g):
        src = table_hbm.at[idx_v.at[g]]
        return pltpu.async_copy(src, rows_v.at[g % NB], gsem)

    for cp in x_cp[: 2 * BATCH]:
        cp.wait()
    g_cp = [fire_gather(0), fire_gather(1)]
    for cp in x_cp[2 * BATCH:]:
        cp.wait()
    s_cp = []

    for g in range(NG):
        pe_cp[g].wait()
        if g + 2 < NG:
            pe_cp.append(fire_pe(g + 2))
        g_cp[g].wait()
        if g + 2 < NG:
            if g >= 1:
                for cp in s_cp[g - 1]:
                    cp.wait()
            g_cp.append(fire_gather(g + 2))

        slot = g % NB
        rb = rows_v.at[slot]
        pb = pe_v.at[slot]

        def body(j, carry):
            for r in range(CHUNK):
                pv = pb[r, pl.ds(j * LANES, LANES)]
                for b in range(BATCH):
                    sl = (b * CHUNK + r, pl.ds(j * LANES, LANES))
                    rb[sl] = rb[sl] * SCALE + pv
            return carry

        lax.fori_loop(0, D_MODEL // LANES, body, 0)

        cps = []
        for b in range(BATCH):
            src = rows_v.at[slot, pl.ds(b * CHUNK, CHUNK)]
            dst = out_hbm.at[b, pl.ds(p0 + g * CHUNK, CHUNK)]
            cps.append(pltpu.async_copy(src, dst, ssem))
        s_cp.append(cps)

    for g in range(NG - 3, NG):
        for cp in s_cp[g]:
            cp.wait()


def kernel(x, table, pos_enc):
    return _emb_kernel(x.astype(jnp.int32), table, pos_enc)
